# baseline (speedup 1.0000x reference)
import functools

import numpy as np
import jax
import jax.numpy as jnp
from jax import lax
from jax.experimental import pallas as pl
from jax.experimental.pallas import tpu as pltpu
from jax.experimental.pallas import tpu_sc as plsc

_F = 26
_D = 64
_B = 16384
_V = 2600000

_NC, _NS = 2, 16
_NW = _NC * _NS

_BPW = _B // _NW
_C = 32
_NCH = _BPW // _C
_RPC = _C * _F
_IPW = _BPW * _F
_SUB = (128, 128, 128, 128, 128, 128, 64)

_W = 256
_NSLAB = 2599936 // _W
_TAIL0 = _NSLAB * _W


def _col_offsets() -> np.ndarray:
    perm = np.arange(_F)
    np.random.RandomState(0).shuffle(perm)
    inv = np.empty(_F, dtype=np.int64)
    inv[perm] = np.arange(_F)
    return (inv * 100000).astype(np.int32)


_COLOFS = _col_offsets()


def _make_transpose_kernel():
    mesh = plsc.VectorSubcoreMesh(core_axis_name="c", subcore_axis_name="s")

    @functools.partial(
        pl.kernel,
        mesh=mesh,
        out_type=jax.ShapeDtypeStruct((_V // 2, 128), jnp.float32),
        scratch_types=[
            pltpu.VMEM((_D, _W), jnp.float32),
            pltpu.VMEM((_W // 2, 128), jnp.float32),
            pltpu.SemaphoreType.DMA,
        ],
        compiler_params=pltpu.CompilerParams(
            use_tc_tiling_on_sc=True, needs_layout_passes=False
        ),
    )
    def t_kernel(wt_hbm, aux_hbm, out_hbm, slab_v, ot_v, sem):
        cid = lax.axis_index("c")
        sid = lax.axis_index("s")
        wid = sid * _NC + cid
        lane = lax.broadcasted_iota(jnp.int32, (16,), 0)

        def transpose_rows(src_v, width):
            def row_body(r, carry):
                orow = r // 2
                ocol = (r % 2) * 64
                for d in range(4):
                    vals = plsc.load_gather(src_v, [lane + d * 16, jnp.broadcast_to(r, (16,))])
                    ot_v[orow, pl.ds(ocol + d * 16, 16)] = vals
                return carry

            lax.fori_loop(0, width, row_body, 0)

        def do_slab(i0, width):
            i0 = pl.multiple_of(i0, 128)
            pltpu.sync_copy(wt_hbm.at[:, pl.ds(i0, width)], slab_v.at[:, pl.ds(0, width)])
            transpose_rows(slab_v, width)
            pltpu.sync_copy(
                ot_v.at[pl.ds(0, width // 2), :],
                out_hbm.at[pl.ds(pl.multiple_of(i0 // 2, 64), width // 2), :],
            )

        nmine = (_NSLAB - wid + _NW - 1) // _NW

        def my_slab(k, carry):
            do_slab((wid + k * _NW) * _W, _W)
            return carry

        lax.fori_loop(0, nmine, my_slab, 0)

        @pl.when(wid == _NW - 1)
        def _tail():
            pltpu.sync_copy(aux_hbm, slab_v.at[:, pl.ds(0, 128)])
            transpose_rows(slab_v, 128)
            pltpu.sync_copy(
                ot_v.at[pl.ds(0, 64), :],
                out_hbm.at[pl.ds((_V - 128) // 2, 64), :],
            )

    return t_kernel


def _make_gather_kernel():
    mesh = plsc.VectorSubcoreMesh(core_axis_name="c", subcore_axis_name="s")

    @functools.partial(
        pl.kernel,
        mesh=mesh,
        out_type=jax.ShapeDtypeStruct((_B, _D), jnp.float32),
        scratch_types=[
            pltpu.VMEM((_IPW,), jnp.int32),
            pltpu.VMEM((_RPC, _D), jnp.float32),
            pltpu.VMEM((_C, _D), jnp.float32),
            pltpu.SemaphoreType.DMA,
        ],
        compiler_params=pltpu.CompilerParams(use_tc_tiling_on_sc=False),
    )
    def g_kernel(idx_hbm, table_hbm, out_hbm, idx_v, rows_v, out_v, sem):
        cid = lax.axis_index("c")
        sid = lax.axis_index("s")
        wid = sid * _NC + cid
        ibase = wid * _IPW
        pltpu.sync_copy(idx_hbm.at[pl.ds(ibase, _IPW)], idx_v)

        for ch in range(_NCH):
            r0 = ch * _RPC
            copies = []
            off = 0
            for n in _SUB:
                copies.append(
                    pltpu.async_copy(
                        table_hbm.at[idx_v.at[pl.ds(r0 + off, n)]],
                        rows_v.at[pl.ds(off, n)],
                        sem,
                    )
                )
                off += n
            for cp in copies:
                cp.wait()

            def bag_body(i, carry):
                base = i * _F
                acc = [rows_v[base, pl.ds(d * 16, 16)] for d in range(4)]
                for f in range(1, _F):
                    r = base + f
                    for d in range(4):
                        acc[d] = acc[d] + rows_v[r, pl.ds(d * 16, 16)]
                for d in range(4):
                    out_v[i, pl.ds(d * 16, 16)] = acc[d]
                return carry

            lax.fori_loop(0, _C, bag_body, 0)
            pltpu.sync_copy(out_v, out_hbm.at[pl.ds(wid * _BPW + ch * _C, _C)])

    return g_kernel


_t_kernel = _make_transpose_kernel()
_g_kernel = _make_gather_kernel()


@jax.jit
def kernel(x, embed_weight):
    idx = (x + jnp.asarray(_COLOFS)[None, :]).reshape(-1)
    wt = embed_weight.T
    wlin = _t_kernel(wt, wt[:, _V - 128:])
    table = wlin.reshape(_V, _D)
    return _g_kernel(idx, table)


# speedup vs baseline: 1.1871x; 1.1871x over previous
import functools

import numpy as np
import jax
import jax.numpy as jnp
from jax import lax
from jax.experimental import pallas as pl
from jax.experimental.pallas import tpu as pltpu
from jax.experimental.pallas import tpu_sc as plsc

_F = 26
_D = 64
_B = 16384
_V = 2600000

_NC, _NS = 2, 16
_NW = _NC * _NS

_BPW = _B // _NW
_C = 32
_NCH = _BPW // _C
_RPC = _C * _F
_IPW = _BPW * _F
_SUB = (128, 128, 128, 128, 128, 128, 64)

_W = 256
_NSLAB = 2599936 // _W
_TAIL0 = _NSLAB * _W


def _col_offsets() -> np.ndarray:
    perm = np.arange(_F)
    np.random.RandomState(0).shuffle(perm)
    inv = np.empty(_F, dtype=np.int64)
    inv[perm] = np.arange(_F)
    return (inv * 100000).astype(np.int32)


_COLOFS = _col_offsets()


def _make_transpose_kernel():
    mesh = plsc.VectorSubcoreMesh(core_axis_name="c", subcore_axis_name="s")

    @functools.partial(
        pl.kernel,
        mesh=mesh,
        out_type=jax.ShapeDtypeStruct((_V // 2, 128), jnp.float32),
        scratch_types=[
            pltpu.VMEM((_D, _W), jnp.float32),
            pltpu.VMEM((_W // 2, 128), jnp.float32),
            pltpu.SemaphoreType.DMA,
        ],
        compiler_params=pltpu.CompilerParams(
            use_tc_tiling_on_sc=True, needs_layout_passes=False
        ),
    )
    def t_kernel(wt_hbm, aux_hbm, out_hbm, slab_v, ot_v, sem):
        cid = lax.axis_index("c")
        sid = lax.axis_index("s")
        wid = sid * _NC + cid
        lane = lax.broadcasted_iota(jnp.int32, (16,), 0)

        def transpose_rows(src_v, width):
            def grp_body(g, carry):
                r0 = pl.multiple_of(g * 16, 16)
                ivec = r0 + lane
                orow = lax.shift_right_logical(ivec, 1)
                ocol0 = lax.shift_left(jnp.bitwise_and(ivec, 1), 6)
                for c in range(_D):
                    vals = src_v[c, pl.ds(r0, 16)]
                    plsc.store_scatter(ot_v, [orow, ocol0 + c], vals)
                return carry

            lax.fori_loop(0, width // 16, grp_body, 0)

        def do_slab(i0, width):
            i0 = pl.multiple_of(i0, 128)
            pltpu.sync_copy(wt_hbm.at[:, pl.ds(i0, width)], slab_v.at[:, pl.ds(0, width)])
            transpose_rows(slab_v, width)
            pltpu.sync_copy(
                ot_v.at[pl.ds(0, width // 2), :],
                out_hbm.at[pl.ds(pl.multiple_of(i0 // 2, 64), width // 2), :],
            )

        nmine = (_NSLAB - wid + _NW - 1) // _NW

        def my_slab(k, carry):
            do_slab((wid + k * _NW) * _W, _W)
            return carry

        lax.fori_loop(0, nmine, my_slab, 0)

        @pl.when(wid == _NW - 1)
        def _tail():
            pltpu.sync_copy(aux_hbm, slab_v.at[:, pl.ds(0, 128)])
            transpose_rows(slab_v, 128)
            pltpu.sync_copy(
                ot_v.at[pl.ds(0, 64), :],
                out_hbm.at[pl.ds((_V - 128) // 2, 64), :],
            )

    return t_kernel


def _make_gather_kernel():
    mesh = plsc.VectorSubcoreMesh(core_axis_name="c", subcore_axis_name="s")

    @functools.partial(
        pl.kernel,
        mesh=mesh,
        out_type=jax.ShapeDtypeStruct((_B, _D), jnp.float32),
        scratch_types=[
            pltpu.VMEM((_IPW,), jnp.int32),
            pltpu.VMEM((_RPC, _D), jnp.float32),
            pltpu.VMEM((_C, _D), jnp.float32),
            pltpu.SemaphoreType.DMA,
        ],
        compiler_params=pltpu.CompilerParams(use_tc_tiling_on_sc=False),
    )
    def g_kernel(idx_hbm, table_hbm, out_hbm, idx_v, rows_v, out_v, sem):
        cid = lax.axis_index("c")
        sid = lax.axis_index("s")
        wid = sid * _NC + cid
        ibase = wid * _IPW
        pltpu.sync_copy(idx_hbm.at[pl.ds(ibase, _IPW)], idx_v)

        for ch in range(_NCH):
            r0 = ch * _RPC
            copies = []
            off = 0
            for n in _SUB:
                copies.append(
                    pltpu.async_copy(
                        table_hbm.at[idx_v.at[pl.ds(r0 + off, n)]],
                        rows_v.at[pl.ds(off, n)],
                        sem,
                    )
                )
                off += n
            for cp in copies:
                cp.wait()

            def bag_body(i, carry):
                base = i * _F
                acc = [rows_v[base, pl.ds(d * 16, 16)] for d in range(4)]
                for f in range(1, _F):
                    r = base + f
                    for d in range(4):
                        acc[d] = acc[d] + rows_v[r, pl.ds(d * 16, 16)]
                for d in range(4):
                    out_v[i, pl.ds(d * 16, 16)] = acc[d]
                return carry

            lax.fori_loop(0, _C, bag_body, 0)
            pltpu.sync_copy(out_v, out_hbm.at[pl.ds(wid * _BPW + ch * _C, _C)])

    return g_kernel


_t_kernel = _make_transpose_kernel()
_g_kernel = _make_gather_kernel()


@jax.jit
def kernel(x, embed_weight):
    idx = (x + jnp.asarray(_COLOFS)[None, :]).reshape(-1)
    wt = embed_weight.T
    wlin = _t_kernel(wt, wt[:, _V - 128:])
    table = wlin.reshape(_V, _D)
    return _g_kernel(idx, table)


# speedup vs baseline: 1.5439x; 1.3006x over previous
import functools

import numpy as np
import jax
import jax.numpy as jnp
from jax import lax
from jax.experimental import pallas as pl
from jax.experimental.pallas import tpu as pltpu
from jax.experimental.pallas import tpu_sc as plsc

_F = 26
_D = 64
_B = 16384
_V = 2600000

_NC, _NS = 2, 16
_NW = _NC * _NS

_BPW = _B // _NW
_C = 32
_NCH = _BPW // _C
_RPC = _C * _F
_IPW = _BPW * _F
_SUB = (128, 128, 128, 128, 128, 128, 64)

_W = 256
_NSLAB = 2599936 // _W
_TAIL0 = _NSLAB * _W


def _col_offsets() -> np.ndarray:
    perm = np.arange(_F)
    np.random.RandomState(0).shuffle(perm)
    inv = np.empty(_F, dtype=np.int64)
    inv[perm] = np.arange(_F)
    return (inv * 100000).astype(np.int32)


_COLOFS = _col_offsets()


def _make_transpose_kernel():
    mesh = plsc.VectorSubcoreMesh(core_axis_name="c", subcore_axis_name="s")

    @functools.partial(
        pl.kernel,
        mesh=mesh,
        out_type=jax.ShapeDtypeStruct((_V // 2, 128), jnp.float32),
        scratch_types=[
            pltpu.VMEM((_D, _W), jnp.float32),
            pltpu.VMEM((_W // 2, 128), jnp.float32),
            pltpu.SemaphoreType.DMA,
        ],
        compiler_params=pltpu.CompilerParams(
            use_tc_tiling_on_sc=True, needs_layout_passes=False
        ),
    )
    def t_kernel(wt_hbm, aux_hbm, out_hbm, slab_v, ot_v, sem):
        cid = lax.axis_index("c")
        sid = lax.axis_index("s")
        wid = sid * _NC + cid
        lane = lax.broadcasted_iota(jnp.int32, (16,), 0)

        def transpose_rows(src_v, width):
            def grp_body(g, carry):
                r0 = pl.multiple_of(g * 16, 16)
                ivec = r0 + lane
                orow = lax.shift_right_logical(ivec, 1)
                ocol0 = lax.shift_left(jnp.bitwise_and(ivec, 1), 6)

                @plsc.parallel_loop(0, _D, step=1, unroll=8)
                def cbody(c):
                    vals = src_v[c, pl.ds(r0, 16)]
                    plsc.store_scatter(ot_v, [orow, ocol0 + c], vals)

                return carry

            lax.fori_loop(0, width // 16, grp_body, 0)

        def do_slab(i0, width):
            i0 = pl.multiple_of(i0, 128)
            pltpu.sync_copy(wt_hbm.at[:, pl.ds(i0, width)], slab_v.at[:, pl.ds(0, width)])
            transpose_rows(slab_v, width)
            pltpu.sync_copy(
                ot_v.at[pl.ds(0, width // 2), :],
                out_hbm.at[pl.ds(pl.multiple_of(i0 // 2, 64), width // 2), :],
            )

        nmine = (_NSLAB - wid + _NW - 1) // _NW

        def my_slab(k, carry):
            do_slab((wid + k * _NW) * _W, _W)
            return carry

        lax.fori_loop(0, nmine, my_slab, 0)

        @pl.when(wid == _NW - 1)
        def _tail():
            pltpu.sync_copy(aux_hbm, slab_v.at[:, pl.ds(0, 128)])
            transpose_rows(slab_v, 128)
            pltpu.sync_copy(
                ot_v.at[pl.ds(0, 64), :],
                out_hbm.at[pl.ds((_V - 128) // 2, 64), :],
            )

    return t_kernel


def _make_gather_kernel():
    mesh = plsc.VectorSubcoreMesh(core_axis_name="c", subcore_axis_name="s")

    @functools.partial(
        pl.kernel,
        mesh=mesh,
        out_type=jax.ShapeDtypeStruct((_B, _D), jnp.float32),
        scratch_types=[
            pltpu.VMEM((_IPW,), jnp.int32),
            pltpu.VMEM((_RPC, _D), jnp.float32),
            pltpu.VMEM((_C, _D), jnp.float32),
            pltpu.SemaphoreType.DMA,
        ],
        compiler_params=pltpu.CompilerParams(use_tc_tiling_on_sc=False),
    )
    def g_kernel(idx_hbm, table_hbm, out_hbm, idx_v, rows_v, out_v, sem):
        cid = lax.axis_index("c")
        sid = lax.axis_index("s")
        wid = sid * _NC + cid
        ibase = wid * _IPW
        pltpu.sync_copy(idx_hbm.at[pl.ds(ibase, _IPW)], idx_v)

        for ch in range(_NCH):
            r0 = ch * _RPC
            copies = []
            off = 0
            for n in _SUB:
                copies.append(
                    pltpu.async_copy(
                        table_hbm.at[idx_v.at[pl.ds(r0 + off, n)]],
                        rows_v.at[pl.ds(off, n)],
                        sem,
                    )
                )
                off += n
            for cp in copies:
                cp.wait()

            def bag_body(i, carry):
                base = i * _F
                acc = [rows_v[base, pl.ds(d * 16, 16)] for d in range(4)]
                for f in range(1, _F):
                    r = base + f
                    for d in range(4):
                        acc[d] = acc[d] + rows_v[r, pl.ds(d * 16, 16)]
                for d in range(4):
                    out_v[i, pl.ds(d * 16, 16)] = acc[d]
                return carry

            lax.fori_loop(0, _C, bag_body, 0)
            pltpu.sync_copy(out_v, out_hbm.at[pl.ds(wid * _BPW + ch * _C, _C)])

    return g_kernel


_t_kernel = _make_transpose_kernel()
_g_kernel = _make_gather_kernel()


@jax.jit
def kernel(x, embed_weight):
    idx = (x + jnp.asarray(_COLOFS)[None, :]).reshape(-1)
    wt = embed_weight.T
    wlin = _t_kernel(wt, wt[:, _V - 128:])
    table = wlin.reshape(_V, _D)
    return _g_kernel(idx, table)


# speedup vs baseline: 6.6555x; 4.3108x over previous
import functools

import numpy as np
import jax
import jax.numpy as jnp
from jax import lax
from jax.experimental import pallas as pl
from jax.experimental.pallas import tpu as pltpu
from jax.experimental.pallas import tpu_sc as plsc

_F = 26
_D = 64
_B = 16384
_V = 2600000

_NC, _NS = 2, 16
_NW = _NC * _NS

_BPW = _B // _NW
_C = 32
_NCH = _BPW // _C
_RPC = _C * _F
_IPW = _BPW * _F
_SUB = (128, 128, 128, 128, 128, 128, 64)

_W = 256
_NSLAB = 2599936 // _W
_TAIL0 = _NSLAB * _W


def _col_offsets() -> np.ndarray:
    perm = np.arange(_F)
    np.random.RandomState(0).shuffle(perm)
    inv = np.empty(_F, dtype=np.int64)
    inv[perm] = np.arange(_F)
    return (inv * 100000).astype(np.int32)


_COLOFS = _col_offsets()


def _make_transpose_kernel():
    mesh = plsc.VectorSubcoreMesh(core_axis_name="c", subcore_axis_name="s")

    @functools.partial(
        pl.kernel,
        mesh=mesh,
        out_type=jax.ShapeDtypeStruct((_V // 2, 128), jnp.float32),
        scratch_types=[
            pltpu.VMEM((2, _D, _W), jnp.float32),
            pltpu.VMEM((2, _W // 2, 128), jnp.float32),
            pltpu.SemaphoreType.DMA((4,)),
        ],
        compiler_params=pltpu.CompilerParams(
            use_tc_tiling_on_sc=True, needs_layout_passes=False
        ),
    )
    def t_kernel(wt_hbm, aux_hbm, out_hbm, slab_v, ot_v, sems):
        cid = lax.axis_index("c")
        sid = lax.axis_index("s")
        wid = sid * _NC + cid
        lane = lax.broadcasted_iota(jnp.int32, (16,), 0)

        def transpose_rows(src_v, dst_v, width):
            def j_body(j, carry):
                ivec = j * 16 + lane
                orow = lax.shift_right_logical(ivec, 1)
                pbit = lax.shift_left(jnp.bitwise_and(ivec, 1), 6)

                @plsc.parallel_loop(0, _D, step=1, unroll=8)
                def d_body(d):
                    cvec = jnp.bitwise_and(ivec + d, 63)
                    vals = plsc.load_gather(src_v, [cvec, ivec])
                    plsc.store_scatter(dst_v, [orow, pbit + cvec], vals)

                return carry

            lax.fori_loop(0, width // 16, j_body, 0)

        def slab_origin(k):
            return pl.multiple_of((wid + k * _NW) * _W, 128)

        def in_desc(k, b):
            return pltpu.make_async_copy(
                wt_hbm.at[:, pl.ds(slab_origin(k), _W)], slab_v.at[b], sems.at[b]
            )

        def out_desc(k, b):
            return pltpu.make_async_copy(
                ot_v.at[b],
                out_hbm.at[pl.ds(pl.multiple_of(slab_origin(k) // 2, 64), _W // 2), :],
                sems.at[2 + b],
            )

        nmine = (_NSLAB - wid + _NW - 1) // _NW
        in_desc(0, 0).start()

        def my_slab(k, carry):
            b = jnp.bitwise_and(k, 1)

            @pl.when(k + 1 < nmine)
            def _():
                in_desc(k + 1, 1 - b).start()

            in_desc(k, b).wait()

            @pl.when(k >= 2)
            def _():
                out_desc(k - 2, b).wait()

            transpose_rows(slab_v.at[b], ot_v.at[b], _W)
            out_desc(k, b).start()
            return carry

        lax.fori_loop(0, nmine, my_slab, 0)
        out_desc(nmine - 2, jnp.bitwise_and(nmine - 2, 1)).wait()
        out_desc(nmine - 1, jnp.bitwise_and(nmine - 1, 1)).wait()

        @pl.when(wid == _NW - 1)
        def _tail():
            pltpu.sync_copy(aux_hbm, slab_v.at[0, :, pl.ds(0, 128)])
            transpose_rows(slab_v.at[0], ot_v.at[0], 128)
            pltpu.sync_copy(
                ot_v.at[0, pl.ds(0, 64), :],
                out_hbm.at[pl.ds((_V - 128) // 2, 64), :],
            )

    return t_kernel


def _make_gather_kernel():
    mesh = plsc.VectorSubcoreMesh(core_axis_name="c", subcore_axis_name="s")

    @functools.partial(
        pl.kernel,
        mesh=mesh,
        out_type=jax.ShapeDtypeStruct((_B, _D), jnp.float32),
        scratch_types=[
            pltpu.VMEM((_IPW,), jnp.int32),
            pltpu.VMEM((_RPC, _D), jnp.float32),
            pltpu.VMEM((_C, _D), jnp.float32),
            pltpu.SemaphoreType.DMA,
        ],
        compiler_params=pltpu.CompilerParams(use_tc_tiling_on_sc=False),
    )
    def g_kernel(idx_hbm, table_hbm, out_hbm, idx_v, rows_v, out_v, sem):
        cid = lax.axis_index("c")
        sid = lax.axis_index("s")
        wid = sid * _NC + cid
        ibase = wid * _IPW
        pltpu.sync_copy(idx_hbm.at[pl.ds(ibase, _IPW)], idx_v)

        for ch in range(_NCH):
            r0 = ch * _RPC
            copies = []
            off = 0
            for n in _SUB:
                copies.append(
                    pltpu.async_copy(
                        table_hbm.at[idx_v.at[pl.ds(r0 + off, n)]],
                        rows_v.at[pl.ds(off, n)],
                        sem,
                    )
                )
                off += n
            for cp in copies:
                cp.wait()

            def bag_body(i, carry):
                base = i * _F
                acc = [rows_v[base, pl.ds(d * 16, 16)] for d in range(4)]
                for f in range(1, _F):
                    r = base + f
                    for d in range(4):
                        acc[d] = acc[d] + rows_v[r, pl.ds(d * 16, 16)]
                for d in range(4):
                    out_v[i, pl.ds(d * 16, 16)] = acc[d]
                return carry

            lax.fori_loop(0, _C, bag_body, 0)
            pltpu.sync_copy(out_v, out_hbm.at[pl.ds(wid * _BPW + ch * _C, _C)])

    return g_kernel


_t_kernel = _make_transpose_kernel()
_g_kernel = _make_gather_kernel()


@jax.jit
def kernel(x, embed_weight):
    idx = (x + jnp.asarray(_COLOFS)[None, :]).reshape(-1)
    wt = embed_weight.T
    wlin = _t_kernel(wt, wt[:, _V - 128:])
    table = wlin.reshape(_V, _D)
    return _g_kernel(idx, table)


# speedup vs baseline: 7.0749x; 1.0630x over previous
import functools

import numpy as np
import jax
import jax.numpy as jnp
from jax import lax
from jax.experimental import pallas as pl
from jax.experimental.pallas import tpu as pltpu
from jax.experimental.pallas import tpu_sc as plsc

_F = 26
_D = 64
_B = 16384
_V = 2600000

_NC, _NS = 2, 16
_NW = _NC * _NS

_BPW = _B // _NW
_C = 32
_NCH = _BPW // _C
_RPC = _C * _F
_IPW = _BPW * _F
_SUB = (128, 128, 128, 128, 128, 128, 64)

_W = 256
_NSLAB = 2599936 // _W
_TAIL0 = _NSLAB * _W


def _col_offsets() -> np.ndarray:
    perm = np.arange(_F)
    np.random.RandomState(0).shuffle(perm)
    inv = np.empty(_F, dtype=np.int64)
    inv[perm] = np.arange(_F)
    return (inv * 100000).astype(np.int32)


_COLOFS = _col_offsets()


def _make_transpose_kernel():
    mesh = plsc.VectorSubcoreMesh(core_axis_name="c", subcore_axis_name="s")

    @functools.partial(
        pl.kernel,
        mesh=mesh,
        out_type=jax.ShapeDtypeStruct((_V // 2, 128), jnp.float32),
        scratch_types=[
            pltpu.VMEM((2, _D, _W), jnp.float32),
            pltpu.VMEM((2, _W // 2, 128), jnp.float32),
            pltpu.SemaphoreType.DMA((4,)),
        ],
        compiler_params=pltpu.CompilerParams(
            use_tc_tiling_on_sc=True, needs_layout_passes=False
        ),
    )
    def t_kernel(wt_hbm, aux_hbm, out_hbm, slab_v, ot_v, sems):
        cid = lax.axis_index("c")
        sid = lax.axis_index("s")
        wid = sid * _NC + cid
        lane = lax.broadcasted_iota(jnp.int32, (16,), 0)

        def transpose_rows(src_v, dst_v, width):
            def j_body(j, carry):
                ivec = j * 16 + lane
                orow = lax.shift_right_logical(ivec, 1)
                pbit = lax.shift_left(jnp.bitwise_and(ivec, 1), 6)

                @plsc.parallel_loop(0, _D, step=1, unroll=8)
                def d_body(d):
                    cvec = jnp.bitwise_and(ivec + d, 63)
                    vals = plsc.load_gather(src_v, [cvec, ivec])
                    plsc.store_scatter(dst_v, [orow, pbit + cvec], vals)

                return carry

            lax.fori_loop(0, width // 16, j_body, 0, unroll=2)

        def slab_origin(k):
            return pl.multiple_of((wid + k * _NW) * _W, 128)

        def in_desc(k, b):
            return pltpu.make_async_copy(
                wt_hbm.at[:, pl.ds(slab_origin(k), _W)], slab_v.at[b], sems.at[b]
            )

        def out_desc(k, b):
            return pltpu.make_async_copy(
                ot_v.at[b],
                out_hbm.at[pl.ds(pl.multiple_of(slab_origin(k) // 2, 64), _W // 2), :],
                sems.at[2 + b],
            )

        nmine = (_NSLAB - wid + _NW - 1) // _NW
        in_desc(0, 0).start()

        def my_slab(k, carry):
            b = jnp.bitwise_and(k, 1)

            @pl.when(k + 1 < nmine)
            def _():
                in_desc(k + 1, 1 - b).start()

            in_desc(k, b).wait()

            @pl.when(k >= 2)
            def _():
                out_desc(k - 2, b).wait()

            transpose_rows(slab_v.at[b], ot_v.at[b], _W)
            out_desc(k, b).start()
            return carry

        lax.fori_loop(0, nmine, my_slab, 0)
        out_desc(nmine - 2, jnp.bitwise_and(nmine - 2, 1)).wait()
        out_desc(nmine - 1, jnp.bitwise_and(nmine - 1, 1)).wait()

        @pl.when(wid == _NW - 1)
        def _tail():
            pltpu.sync_copy(aux_hbm, slab_v.at[0, :, pl.ds(0, 128)])
            transpose_rows(slab_v.at[0], ot_v.at[0], 128)
            pltpu.sync_copy(
                ot_v.at[0, pl.ds(0, 64), :],
                out_hbm.at[pl.ds((_V - 128) // 2, 64), :],
            )

    return t_kernel


def _make_gather_kernel():
    mesh = plsc.VectorSubcoreMesh(core_axis_name="c", subcore_axis_name="s")

    @functools.partial(
        pl.kernel,
        mesh=mesh,
        out_type=jax.ShapeDtypeStruct((_B, _D), jnp.float32),
        scratch_types=[
            pltpu.VMEM((_IPW,), jnp.int32),
            pltpu.VMEM((2, _RPC, _D), jnp.float32),
            pltpu.VMEM((2, _C, _D), jnp.float32),
            pltpu.SemaphoreType.DMA((4,)),
        ],
        compiler_params=pltpu.CompilerParams(use_tc_tiling_on_sc=False),
    )
    def g_kernel(idx_hbm, table_hbm, out_hbm, idx_v, rows_v, out_v, sems):
        cid = lax.axis_index("c")
        sid = lax.axis_index("s")
        wid = sid * _NC + cid
        ibase = wid * _IPW
        pltpu.sync_copy(idx_hbm.at[pl.ds(ibase, _IPW)], idx_v)

        def start_gathers(ch, b):
            r0 = ch * _RPC
            handles = []
            off = 0
            for n in _SUB:
                handles.append(
                    pltpu.async_copy(
                        table_hbm.at[idx_v.at[pl.ds(r0 + off, n)]],
                        rows_v.at[b].at[pl.ds(off, n)],
                        sems.at[b],
                    )
                )
                off += n
            return handles

        pend_in = {0: start_gathers(0, 0), 1: None}
        pend_out = {0: None, 1: None}
        for ch in range(_NCH):
            b = ch % 2
            if ch + 1 < _NCH:
                pend_in[1 - b] = start_gathers(ch + 1, 1 - b)
            for cp in pend_in[b]:
                cp.wait()
            if pend_out[b] is not None:
                pend_out[b].wait()

            def bag_body(i, carry):
                base = i * _F
                acc = [rows_v[b, base, pl.ds(d * 16, 16)] for d in range(4)]
                for f in range(1, _F):
                    r = base + f
                    for d in range(4):
                        acc[d] = acc[d] + rows_v[b, r, pl.ds(d * 16, 16)]
                for d in range(4):
                    out_v[b, i, pl.ds(d * 16, 16)] = acc[d]
                return carry

            lax.fori_loop(0, _C, bag_body, 0)
            pend_out[b] = pltpu.async_copy(
                out_v.at[b], out_hbm.at[pl.ds(wid * _BPW + ch * _C, _C)], sems.at[2 + b]
            )
        pend_out[0].wait()
        pend_out[1].wait()

    return g_kernel


_t_kernel = _make_transpose_kernel()
_g_kernel = _make_gather_kernel()


@jax.jit
def kernel(x, embed_weight):
    idx = (x + jnp.asarray(_COLOFS)[None, :]).reshape(-1)
    wt = embed_weight.T
    wlin = _t_kernel(wt, wt[:, _V - 128:])
    table = wlin.reshape(_V, _D)
    return _g_kernel(idx, table)
